# baseline (speedup 1.0000x reference)
import functools

import jax
import jax.numpy as jnp
from jax import lax
from jax.experimental import pallas as pl
from jax.experimental.pallas import tpu as pltpu
from jax.experimental.pallas import tpu_sc as plsc

N = 10000
D = 128
H = 8
F = 16
TW = D + 2 * H
N_CORES = 2
N_SUB = 16
N_WORK = N_CORES * N_SUB
CH = 64
E_RAW = 320000
NBUF = 3
ISUP = 27
NSUP = 6
CHUNKS = ISUP * NSUP
E_PAD = N_WORK * CH * CHUNKS
EPW = CH * CHUNKS
N_ACC = 10112
ROWS_PER_TILE = N_ACC // N_SUB


def _embed_body(x_ref, w_ref, alr_ref, arr_ref, t_ref, er_ref):
    h = jnp.dot(x_ref[...], w_ref[...], preferred_element_type=jnp.float32)
    t_ref[:, :D] = h
    t_ref[:, D:TW] = jnp.dot(h, alr_ref[...], preferred_element_type=jnp.float32)
    er_ref[...] = jnp.dot(h, arr_ref[...], preferred_element_type=jnp.float32)


def _embed(x, w, alr, arr, rows_blk):
    n = x.shape[0]
    return pl.pallas_call(
        _embed_body,
        grid=(n // rows_blk,),
        in_specs=[
            pl.BlockSpec((rows_blk, D), lambda i: (i, 0)),
            pl.BlockSpec((D, D), lambda i: (0, 0)),
            pl.BlockSpec((D, 2 * H), lambda i: (0, 0)),
            pl.BlockSpec((D, 2 * H), lambda i: (0, 0)),
        ],
        out_specs=[
            pl.BlockSpec((rows_blk, TW), lambda i: (i, 0)),
            pl.BlockSpec((rows_blk, 2 * H), lambda i: (i, 0)),
        ],
        out_shape=[
            jax.ShapeDtypeStruct((n, TW), jnp.float32),
            jax.ShapeDtypeStruct((n, 2 * H), jnp.float32),
        ],
    )(x, w, alr, arr)


def _norm_embed_body(acc_ref, rep_ref, b_ref, w_ref, alr_ref, arr_ref,
                     t_ref, er_ref):
    s = acc_ref[0] + acc_ref[1]
    den = jnp.dot(s[:, D:D + H], rep_ref[...], preferred_element_type=jnp.float32)
    x1 = s[:, :D] / (den + 1e-9) + b_ref[...]
    x1 = jnp.maximum(x1, 0.01 * x1)
    h = jnp.dot(x1, w_ref[...], preferred_element_type=jnp.float32)
    t_ref[:, :D] = h
    t_ref[:, D:TW] = jnp.dot(h, alr_ref[...], preferred_element_type=jnp.float32)
    er_ref[...] = jnp.dot(h, arr_ref[...], preferred_element_type=jnp.float32)


def _norm_embed(acc, rep, b, w, alr, arr, rows_blk):
    n = acc.shape[1]
    return pl.pallas_call(
        _norm_embed_body,
        grid=(n // rows_blk,),
        in_specs=[
            pl.BlockSpec((2, rows_blk, TW), lambda i: (0, i, 0)),
            pl.BlockSpec((H, D), lambda i: (0, 0)),
            pl.BlockSpec((1, D), lambda i: (0, 0)),
            pl.BlockSpec((D, D), lambda i: (0, 0)),
            pl.BlockSpec((D, 2 * H), lambda i: (0, 0)),
            pl.BlockSpec((D, 2 * H), lambda i: (0, 0)),
        ],
        out_specs=[
            pl.BlockSpec((rows_blk, TW), lambda i: (i, 0)),
            pl.BlockSpec((rows_blk, 2 * H), lambda i: (i, 0)),
        ],
        out_shape=[
            jax.ShapeDtypeStruct((n, TW), jnp.float32),
            jax.ShapeDtypeStruct((n, 2 * H), jnp.float32),
        ],
    )(acc, rep, b, w, alr, arr)


def _final_body(acc_ref, rep_ref, b_ref, o_ref):
    s = acc_ref[0] + acc_ref[1]
    den = jnp.dot(s[:, D:D + H], rep_ref[...], preferred_element_type=jnp.float32)
    o_ref[...] = s[:, :D] / (den + 1e-9) + b_ref[...]


def _final(acc, rep, b, rows_blk):
    return pl.pallas_call(
        _final_body,
        grid=(N // rows_blk,),
        in_specs=[
            pl.BlockSpec((2, rows_blk, TW), lambda i: (0, i, 0)),
            pl.BlockSpec((H, D), lambda i: (0, 0)),
            pl.BlockSpec((1, D), lambda i: (0, 0)),
        ],
        out_specs=pl.BlockSpec((rows_blk, D), lambda i: (i, 0)),
        out_shape=jax.ShapeDtypeStruct((N, D), jnp.float32),
    )(acc, rep, b)


@functools.cache
def _make_sc_edge():
    mesh = plsc.VectorSubcoreMesh(core_axis_name="c", subcore_axis_name="s")
    return functools.partial(
        pl.kernel,
        mesh=mesh,
        compiler_params=pltpu.CompilerParams(use_tc_tiling_on_sc=False),
        out_type=jax.ShapeDtypeStruct((N_CORES, N_ACC, TW), jnp.float32),
        scratch_types=[
            pltpu.VMEM((2, ISUP, CH), jnp.int32),
            pltpu.VMEM((2, ISUP, CH), jnp.int32),
            pltpu.VMEM((NBUF, CH, TW), jnp.float32),
            pltpu.VMEM((NBUF, CH, 2 * H), jnp.float32),
            pltpu.VMEM_SHARED((N_ACC, TW), jnp.float32),
            pltpu.SemaphoreType.DMA((NBUF,)),
            pltpu.SemaphoreType.DMA((NBUF,)),
            pltpu.SemaphoreType.DMA,
        ],
    )(_sc_edge_body)


def _sc_edge_body(t_hbm, er_hbm, src_hbm, dst_hbm, zero_hbm, out_hbm,
                  idx_s, idx_d, rows, errs, acc, gsem, ssem, isem):
    c = lax.axis_index("c")
    s = lax.axis_index("s")
    pltpu.sync_copy(zero_hbm, acc.at[pl.ds(s * ROWS_PER_TILE, ROWS_PER_TILE)])
    plsc.subcore_barrier()

    w0 = (c * N_SUB + s) * CHUNKS

    def load_idx(sup, slot):
        pltpu.async_copy(src_hbm.at[pl.ds(w0 + sup * ISUP, ISUP)],
                         idx_s.at[slot], isem)
        pltpu.async_copy(dst_hbm.at[pl.ds(w0 + sup * ISUP, ISUP)],
                         idx_d.at[slot], isem)

    def wait_idx(slot):
        pltpu.make_async_copy(src_hbm.at[pl.ds(w0, ISUP)], idx_s.at[slot],
                              isem).wait()
        pltpu.make_async_copy(dst_hbm.at[pl.ds(w0, ISUP)], idx_d.at[slot],
                              isem).wait()

    def start_gather(slot, row, b):
        pltpu.async_copy(t_hbm.at[idx_s.at[slot, row]], rows.at[b], gsem.at[b])
        pltpu.async_copy(er_hbm.at[idx_d.at[slot, row]], errs.at[b],
                         gsem.at[b])

    def wait_gather(b):
        pltpu.make_async_copy(t_hbm.at[idx_s.at[0, 0]], rows.at[b],
                              gsem.at[b]).wait()
        pltpu.make_async_copy(er_hbm.at[idx_d.at[0, 0]], errs.at[b],
                              gsem.at[b]).wait()

    def start_scatter(slot, row, b):
        pltpu.async_copy(rows.at[b], acc.at[idx_d.at[slot, row]], ssem.at[b],
                         add=True)

    def wait_scatter(b):
        pltpu.make_async_copy(rows.at[b], acc.at[idx_d.at[0, 0]],
                              ssem.at[b]).wait()

    def compute(b):
        @plsc.parallel_loop(0, CH, unroll=8)
        def edge_body(e):
            ev = rows[b, e, pl.ds(D, 16)] + errs[b, e, :]
            ev = jnp.maximum(ev, 0.2 * ev)
            wv = jnp.exp(ev)
            rows[b, e, pl.ds(D, 16)] = wv
            for hh in range(H):
                bb = lax.broadcast(wv[hh], (16,))
                rows[b, e, pl.ds(hh * F, F)] = rows[b, e, pl.ds(hh * F, F)] * bb

    pltpu.sync_copy(src_hbm.at[pl.ds(w0, ISUP)], idx_s.at[0])
    pltpu.sync_copy(dst_hbm.at[pl.ds(w0, ISUP)], idx_d.at[0])
    start_gather(0, 0, 0)
    start_gather(0, 1, 1)

    def chunk_body(i, carry):
        j = i // ISUP
        t = i - j * ISUP
        cs = j % 2
        ns = 1 - cs
        b = i % NBUF
        b2 = (i + 2) % NBUF
        more_sups = j < NSUP - 1

        @pl.when(jnp.logical_and(t == 1, more_sups))
        def _():
            load_idx(j + 1, ns)

        wait_gather(b)
        compute(b)
        start_scatter(cs, t, b)

        @pl.when(i > 0)
        def _():
            wait_scatter(b2)

        @pl.when(jnp.logical_and(t == ISUP - 2, more_sups))
        def _():
            wait_idx(ns)

        tp = t + 2
        wrap = tp >= ISUP
        g_slot = jnp.where(wrap, ns, cs)
        g_row = jnp.where(wrap, tp - ISUP, tp)

        @pl.when(jnp.logical_or(jnp.logical_not(wrap), more_sups))
        def _():
            start_gather(g_slot, g_row, b2)

        return carry

    lax.fori_loop(0, CHUNKS, chunk_body, 0)
    wait_scatter((CHUNKS - 1) % NBUF)
    plsc.subcore_barrier()
    pltpu.sync_copy(acc.at[pl.ds(s * ROWS_PER_TILE, ROWS_PER_TILE)],
                    out_hbm.at[c, pl.ds(s * ROWS_PER_TILE, ROWS_PER_TILE)])


def _block_diag(a):
    eye = jnp.eye(H, dtype=jnp.float32)
    return (a[:, :, None] * eye[:, None, :]).reshape(D, H)


def kernel(n_feat, edge_index, e_feat, W1, al1, ar1, b1, W2, al2, ar2, b2):
    del e_feat
    ei = edge_index.astype(jnp.int32)
    pad_e = E_PAD - E_RAW
    src = jnp.concatenate([ei[0], jnp.zeros((pad_e,), jnp.int32)])
    src = src.reshape(E_PAD // CH, CH)
    dst = jnp.concatenate([ei[1], jnp.full((pad_e,), N, jnp.int32)])
    dst = dst.reshape(E_PAD // CH, CH)
    zero_blk = jnp.zeros((ROWS_PER_TILE, TW), jnp.float32)
    rep = jnp.repeat(jnp.eye(H, dtype=jnp.float32), F, axis=1)

    alr1 = jnp.concatenate([_block_diag(al1), _block_diag(ar1)], axis=1)
    arr1 = jnp.concatenate([_block_diag(ar1), _block_diag(ar1)], axis=1)
    alr2 = jnp.concatenate([_block_diag(al2), _block_diag(ar2)], axis=1)
    arr2 = jnp.concatenate([_block_diag(ar2), _block_diag(ar2)], axis=1)

    x = jnp.pad(n_feat, ((0, N_ACC - N), (0, 0)))
    t1, er1 = _embed(x, W1, alr1, arr1, rows_blk=2528)
    sc_edge = _make_sc_edge()
    acc1 = sc_edge(t1, er1, src, dst, zero_blk)
    t2, er2 = _norm_embed(acc1, rep, b1.reshape(1, D), W2, alr2, arr2,
                          rows_blk=2528)
    acc2 = sc_edge(t2, er2, src, dst, zero_blk)
    return _final(acc2, rep, b2.reshape(1, D), rows_blk=2000)


# speedup vs baseline: 1.3467x; 1.3467x over previous
import functools

import jax
import jax.numpy as jnp
import numpy as np
from jax import lax
from jax.experimental import pallas as pl
from jax.experimental.pallas import tpu as pltpu
from jax.experimental.pallas import tpu_sc as plsc

N = 10000
D = 128
H = 8
F = 16
TW = D + 2 * H
TM = 160
N_CORES = 2
N_SUB = 16
N_WORK = N_CORES * N_SUB
CH = 64
E_RAW = 320000
NBUF = 3
SBUF = 2
ISUP = 9
NSUP = 18
CHUNKS = ISUP * NSUP
E_PAD = N_WORK * CH * CHUNKS
EPW = CH * CHUNKS
N_ACC = 10112
ROWS_PER_TILE = N_ACC // N_SUB

_P = np.zeros((TW, TM), np.float32)
for _m in range(TM):
    _k, _r = divmod(_m, 32)
    if _k < 4:
        _P[32 * _k + 16 * (_r % 2) + _r // 2, _m] = 1.0
    else:
        _P[D + _r // 2, _m] = 1.0
_Q = np.zeros((2 * H, 4 * H), np.float32)
for _j in range(2 * H):
    _Q[_j, 2 * _j] = 1.0
    _Q[_j, 2 * _j + 1] = 1.0


def _embed_body(x_ref, w_ref, g_ref, ger_ref, t_ref, er_ref):
    h = jnp.dot(x_ref[...], w_ref[...], preferred_element_type=jnp.float32)
    t_ref[...] = jnp.dot(h, g_ref[...],
                         preferred_element_type=jnp.float32).astype(jnp.bfloat16)
    er_ref[...] = jnp.dot(h, ger_ref[...],
                          preferred_element_type=jnp.float32).astype(jnp.bfloat16)


def _embed(x, w, g, ger, rows_blk):
    n = x.shape[0]
    return pl.pallas_call(
        _embed_body,
        grid=(n // rows_blk,),
        in_specs=[
            pl.BlockSpec((rows_blk, D), lambda i: (i, 0)),
            pl.BlockSpec((D, D), lambda i: (0, 0)),
            pl.BlockSpec((D, TM), lambda i: (0, 0)),
            pl.BlockSpec((D, 4 * H), lambda i: (0, 0)),
        ],
        out_specs=[
            pl.BlockSpec((rows_blk, TM), lambda i: (i, 0)),
            pl.BlockSpec((rows_blk, 4 * H), lambda i: (i, 0)),
        ],
        out_shape=[
            jax.ShapeDtypeStruct((n, TM), jnp.bfloat16),
            jax.ShapeDtypeStruct((n, 4 * H), jnp.bfloat16),
        ],
    )(x, w, g, ger)


def _norm_embed_body(acc_ref, rep_ref, b_ref, w_ref, g_ref, ger_ref,
                     t_ref, er_ref):
    s = acc_ref[0] + acc_ref[1]
    den = jnp.dot(s[:, D:D + H], rep_ref[...], preferred_element_type=jnp.float32)
    x1 = s[:, :D] / (den + 1e-9) + b_ref[...]
    x1 = jnp.maximum(x1, 0.01 * x1)
    h = jnp.dot(x1, w_ref[...], preferred_element_type=jnp.float32)
    t_ref[...] = jnp.dot(h, g_ref[...],
                         preferred_element_type=jnp.float32).astype(jnp.bfloat16)
    er_ref[...] = jnp.dot(h, ger_ref[...],
                          preferred_element_type=jnp.float32).astype(jnp.bfloat16)


def _norm_embed(acc, rep, b, w, g, ger, rows_blk):
    n = acc.shape[1]
    return pl.pallas_call(
        _norm_embed_body,
        grid=(n // rows_blk,),
        in_specs=[
            pl.BlockSpec((2, rows_blk, TW), lambda i: (0, i, 0)),
            pl.BlockSpec((H, D), lambda i: (0, 0)),
            pl.BlockSpec((1, D), lambda i: (0, 0)),
            pl.BlockSpec((D, D), lambda i: (0, 0)),
            pl.BlockSpec((D, TM), lambda i: (0, 0)),
            pl.BlockSpec((D, 4 * H), lambda i: (0, 0)),
        ],
        out_specs=[
            pl.BlockSpec((rows_blk, TM), lambda i: (i, 0)),
            pl.BlockSpec((rows_blk, 4 * H), lambda i: (i, 0)),
        ],
        out_shape=[
            jax.ShapeDtypeStruct((n, TM), jnp.bfloat16),
            jax.ShapeDtypeStruct((n, 4 * H), jnp.bfloat16),
        ],
    )(acc, rep, b, w, g, ger)


def _final_body(acc_ref, rep_ref, b_ref, o_ref):
    s = acc_ref[0] + acc_ref[1]
    den = jnp.dot(s[:, D:D + H], rep_ref[...], preferred_element_type=jnp.float32)
    o_ref[...] = s[:, :D] / (den + 1e-9) + b_ref[...]


def _final(acc, rep, b, rows_blk):
    return pl.pallas_call(
        _final_body,
        grid=(N // rows_blk,),
        in_specs=[
            pl.BlockSpec((2, rows_blk, TW), lambda i: (0, i, 0)),
            pl.BlockSpec((H, D), lambda i: (0, 0)),
            pl.BlockSpec((1, D), lambda i: (0, 0)),
        ],
        out_specs=pl.BlockSpec((rows_blk, D), lambda i: (i, 0)),
        out_shape=jax.ShapeDtypeStruct((N, D), jnp.float32),
    )(acc, rep, b)


@functools.cache
def _make_sc_edge():
    mesh = plsc.VectorSubcoreMesh(core_axis_name="c", subcore_axis_name="s")
    return functools.partial(
        pl.kernel,
        mesh=mesh,
        compiler_params=pltpu.CompilerParams(use_tc_tiling_on_sc=False),
        out_type=jax.ShapeDtypeStruct((N_CORES, N_ACC, TW), jnp.float32),
        scratch_types=[
            pltpu.VMEM((2, ISUP, CH), jnp.int32),
            pltpu.VMEM((2, ISUP, CH), jnp.int32),
            pltpu.VMEM((NBUF, CH, TM // 2), jnp.int32),
            pltpu.VMEM((NBUF, CH, 2 * H), jnp.int32),
            pltpu.VMEM((SBUF, CH, TW), jnp.float32),
            pltpu.VMEM_SHARED((N_ACC, TW), jnp.float32),
            pltpu.SemaphoreType.DMA((NBUF,)),
            pltpu.SemaphoreType.DMA((SBUF,)),
            pltpu.SemaphoreType.DMA,
        ],
    )(_sc_edge_body)


def _sc_edge_body(t_hbm, er_hbm, src_hbm, dst_hbm, zero_hbm, out_hbm,
                  idx_s, idx_d, rows, errs, sbuf, acc, gsem, ssem, isem):
    c = lax.axis_index("c")
    s = lax.axis_index("s")
    pltpu.sync_copy(zero_hbm, acc.at[pl.ds(s * ROWS_PER_TILE, ROWS_PER_TILE)])
    plsc.subcore_barrier()

    w0 = (c * N_SUB + s) * CHUNKS

    def load_idx(sup, slot):
        pltpu.async_copy(src_hbm.at[pl.ds(w0 + sup * ISUP, ISUP)],
                         idx_s.at[slot], isem)
        pltpu.async_copy(dst_hbm.at[pl.ds(w0 + sup * ISUP, ISUP)],
                         idx_d.at[slot], isem)

    def wait_idx(slot):
        pltpu.make_async_copy(src_hbm.at[pl.ds(w0, ISUP)], idx_s.at[slot],
                              isem).wait()
        pltpu.make_async_copy(dst_hbm.at[pl.ds(w0, ISUP)], idx_d.at[slot],
                              isem).wait()

    def start_gather(slot, row, b):
        pltpu.async_copy(t_hbm.at[idx_s.at[slot, row]], rows.at[b], gsem.at[b])
        pltpu.async_copy(er_hbm.at[idx_d.at[slot, row]], errs.at[b],
                         gsem.at[b])

    def wait_gather(b):
        pltpu.make_async_copy(t_hbm.at[idx_s.at[0, 0]], rows.at[b],
                              gsem.at[b]).wait()
        pltpu.make_async_copy(er_hbm.at[idx_d.at[0, 0]], errs.at[b],
                              gsem.at[b]).wait()

    def start_scatter(slot, row, p):
        pltpu.async_copy(sbuf.at[p], acc.at[idx_d.at[slot, row]], ssem.at[p],
                         add=True)

    def wait_scatter(p):
        pltpu.make_async_copy(sbuf.at[p], acc.at[idx_d.at[0, 0]],
                              ssem.at[p]).wait()

    def unpack2(vi):
        va = lax.bitcast_convert_type(lax.shift_left(vi, 16), jnp.float32)
        vb = lax.bitcast_convert_type(
            jnp.bitwise_and(vi, jnp.int32(-65536)), jnp.float32)
        return va, vb

    def compute(b, p):
        @plsc.parallel_loop(0, CH, unroll=4)
        def edge_body(e):
            elr, _ = unpack2(rows[b, e, pl.ds(4 * 16, 16)])
            erd, _ = unpack2(errs[b, e, :])
            ev = elr + erd
            ev = jnp.maximum(ev, 0.2 * ev)
            wv = jnp.exp(ev)
            sbuf[p, e, pl.ds(D, 16)] = wv
            for k in range(4):
                ha, hb = unpack2(rows[b, e, pl.ds(16 * k, 16)])
                ba = lax.broadcast(wv[2 * k], (16,))
                bb = lax.broadcast(wv[2 * k + 1], (16,))
                sbuf[p, e, pl.ds(F * 2 * k, F)] = ha * ba
                sbuf[p, e, pl.ds(F * (2 * k + 1), F)] = hb * bb

    pltpu.sync_copy(src_hbm.at[pl.ds(w0, ISUP)], idx_s.at[0])
    pltpu.sync_copy(dst_hbm.at[pl.ds(w0, ISUP)], idx_d.at[0])
    start_gather(0, 0, 0)
    start_gather(0, 1, 1)

    def chunk_body(i, carry):
        j = i // ISUP
        t = i - j * ISUP
        cs = j % 2
        ns = 1 - cs
        b = i % NBUF
        b2 = (i + 2) % NBUF
        p = i % SBUF
        more_sups = j < NSUP - 1

        @pl.when(jnp.logical_and(t == 1, more_sups))
        def _():
            load_idx(j + 1, ns)

        wait_gather(b)

        @pl.when(i > 1)
        def _():
            wait_scatter(p)

        compute(b, p)
        start_scatter(cs, t, p)

        @pl.when(jnp.logical_and(t == ISUP - 2, more_sups))
        def _():
            wait_idx(ns)

        tp = t + 2
        wrap = tp >= ISUP
        g_slot = jnp.where(wrap, ns, cs)
        g_row = jnp.where(wrap, tp - ISUP, tp)

        @pl.when(jnp.logical_or(jnp.logical_not(wrap), more_sups))
        def _():
            start_gather(g_slot, g_row, b2)

        return carry

    lax.fori_loop(0, CHUNKS, chunk_body, 0)
    wait_scatter((CHUNKS - 2) % SBUF)
    wait_scatter((CHUNKS - 1) % SBUF)
    plsc.subcore_barrier()
    pltpu.sync_copy(acc.at[pl.ds(s * ROWS_PER_TILE, ROWS_PER_TILE)],
                    out_hbm.at[c, pl.ds(s * ROWS_PER_TILE, ROWS_PER_TILE)])


def _block_diag(a):
    eye = jnp.eye(H, dtype=jnp.float32)
    return (a[:, :, None] * eye[:, None, :]).reshape(D, H)


def _tables_weights(al, ar):
    alr = jnp.concatenate([_block_diag(al), _block_diag(ar)], axis=1)
    i_alr = jnp.concatenate([jnp.eye(D, dtype=jnp.float32), alr], axis=1)
    g = i_alr @ jnp.asarray(_P)
    arr = jnp.concatenate([_block_diag(ar), _block_diag(ar)], axis=1)
    ger = arr @ jnp.asarray(_Q)
    return g, ger


def kernel(n_feat, edge_index, e_feat, W1, al1, ar1, b1, W2, al2, ar2, b2):
    del e_feat
    ei = edge_index.astype(jnp.int32)
    pad_e = E_PAD - E_RAW
    src = jnp.concatenate([ei[0], jnp.zeros((pad_e,), jnp.int32)])
    src = src.reshape(E_PAD // CH, CH)
    dst = jnp.concatenate([ei[1], jnp.full((pad_e,), N, jnp.int32)])
    dst = dst.reshape(E_PAD // CH, CH)
    zero_blk = jnp.zeros((ROWS_PER_TILE, TW), jnp.float32)
    rep = jnp.repeat(jnp.eye(H, dtype=jnp.float32), F, axis=1)

    g1, ger1 = _tables_weights(al1, ar1)
    g2, ger2 = _tables_weights(al2, ar2)

    def as_i32(t):
        return lax.bitcast_convert_type(
            t.reshape(t.shape[0], t.shape[1] // 2, 2), jnp.int32)

    x = jnp.pad(n_feat, ((0, N_ACC - N), (0, 0)))
    t1, er1 = _embed(x, W1, g1, ger1, rows_blk=2528)
    sc_edge = _make_sc_edge()
    acc1 = sc_edge(as_i32(t1), as_i32(er1), src, dst, zero_blk)
    t2, er2 = _norm_embed(acc1, rep, b1.reshape(1, D), W2, g2, ger2,
                          rows_blk=2528)
    acc2 = sc_edge(as_i32(t2), as_i32(er2), src, dst, zero_blk)
    return _final(acc2, rep, b2.reshape(1, D), rows_blk=2000)


# speedup vs baseline: 1.8097x; 1.3438x over previous
import functools

import jax
import jax.numpy as jnp
import numpy as np
from jax import lax
from jax.experimental import pallas as pl
from jax.experimental.pallas import tpu as pltpu
from jax.experimental.pallas import tpu_sc as plsc

N = 10000
D = 128
H = 8
F = 16
TW = D + 2 * H
TM = 160
N_CORES = 2
N_SUB = 16
N_WORK = N_CORES * N_SUB
CH = 64
E_RAW = 320000
NBUF = 3
SBUF = 2
ISUP = 9
NSUP = 18
CHUNKS = ISUP * NSUP
E_PAD = N_WORK * CH * CHUNKS
EPW = CH * CHUNKS
N_ACC = 10112
ROWS_PER_TILE = N_ACC // N_SUB

_P = np.zeros((TW, TM), np.float32)
for _m in range(TM):
    _k, _r = divmod(_m, 32)
    if _k < 4:
        _P[32 * _k + 16 * (_r % 2) + _r // 2, _m] = 1.0
    else:
        _P[D + _r // 2, _m] = 1.0
_Q = np.zeros((2 * H, 4 * H), np.float32)
for _j in range(2 * H):
    _Q[_j, 2 * _j] = 1.0
    _Q[_j, 2 * _j + 1] = 1.0


def _embed_body(x_ref, w_ref, g_ref, ger_ref, t_ref, er_ref):
    h = jnp.dot(x_ref[...], w_ref[...], preferred_element_type=jnp.float32)
    t_ref[...] = jnp.dot(h, g_ref[...],
                         preferred_element_type=jnp.float32).astype(jnp.bfloat16)
    er_ref[...] = jnp.dot(h, ger_ref[...],
                          preferred_element_type=jnp.float32).astype(jnp.bfloat16)


def _embed(x, w, g, ger, rows_blk):
    n = x.shape[0]
    return pl.pallas_call(
        _embed_body,
        grid=(n // rows_blk,),
        in_specs=[
            pl.BlockSpec((rows_blk, D), lambda i: (i, 0)),
            pl.BlockSpec((D, D), lambda i: (0, 0)),
            pl.BlockSpec((D, TM), lambda i: (0, 0)),
            pl.BlockSpec((D, 4 * H), lambda i: (0, 0)),
        ],
        out_specs=[
            pl.BlockSpec((rows_blk, TM), lambda i: (i, 0)),
            pl.BlockSpec((rows_blk, 4 * H), lambda i: (i, 0)),
        ],
        out_shape=[
            jax.ShapeDtypeStruct((n, TM), jnp.bfloat16),
            jax.ShapeDtypeStruct((n, 4 * H), jnp.bfloat16),
        ],
    )(x, w, g, ger)


def _norm_embed_body(acc_ref, rep_ref, b_ref, w_ref, g_ref, ger_ref,
                     t_ref, er_ref):
    s = acc_ref[0] + acc_ref[1]
    den = jnp.dot(s[:, D:D + H], rep_ref[...], preferred_element_type=jnp.float32)
    x1 = s[:, :D] / (den + 1e-9) + b_ref[...]
    x1 = jnp.maximum(x1, 0.01 * x1)
    h = jnp.dot(x1, w_ref[...], preferred_element_type=jnp.float32)
    t_ref[...] = jnp.dot(h, g_ref[...],
                         preferred_element_type=jnp.float32).astype(jnp.bfloat16)
    er_ref[...] = jnp.dot(h, ger_ref[...],
                          preferred_element_type=jnp.float32).astype(jnp.bfloat16)


def _norm_embed(acc, rep, b, w, g, ger, rows_blk):
    n = acc.shape[1]
    return pl.pallas_call(
        _norm_embed_body,
        grid=(n // rows_blk,),
        in_specs=[
            pl.BlockSpec((2, rows_blk, TW), lambda i: (0, i, 0)),
            pl.BlockSpec((H, D), lambda i: (0, 0)),
            pl.BlockSpec((1, D), lambda i: (0, 0)),
            pl.BlockSpec((D, D), lambda i: (0, 0)),
            pl.BlockSpec((D, TM), lambda i: (0, 0)),
            pl.BlockSpec((D, 4 * H), lambda i: (0, 0)),
        ],
        out_specs=[
            pl.BlockSpec((rows_blk, TM), lambda i: (i, 0)),
            pl.BlockSpec((rows_blk, 4 * H), lambda i: (i, 0)),
        ],
        out_shape=[
            jax.ShapeDtypeStruct((n, TM), jnp.bfloat16),
            jax.ShapeDtypeStruct((n, 4 * H), jnp.bfloat16),
        ],
    )(acc, rep, b, w, g, ger)


def _final_body(acc_ref, rep_ref, b_ref, o_ref):
    s = acc_ref[0] + acc_ref[1]
    den = jnp.dot(s[:, D:D + H], rep_ref[...], preferred_element_type=jnp.float32)
    o_ref[...] = s[:, :D] / (den + 1e-9) + b_ref[...]


def _final(acc, rep, b, rows_blk):
    return pl.pallas_call(
        _final_body,
        grid=(N // rows_blk,),
        in_specs=[
            pl.BlockSpec((2, rows_blk, TW), lambda i: (0, i, 0)),
            pl.BlockSpec((H, D), lambda i: (0, 0)),
            pl.BlockSpec((1, D), lambda i: (0, 0)),
        ],
        out_specs=pl.BlockSpec((rows_blk, D), lambda i: (i, 0)),
        out_shape=jax.ShapeDtypeStruct((N, D), jnp.float32),
    )(acc, rep, b)


@functools.cache
def _make_sc_edge():
    mesh = plsc.VectorSubcoreMesh(core_axis_name="c", subcore_axis_name="s")
    return functools.partial(
        pl.kernel,
        mesh=mesh,
        compiler_params=pltpu.CompilerParams(use_tc_tiling_on_sc=False),
        out_type=jax.ShapeDtypeStruct((N_CORES, N_ACC, TW), jnp.float32),
        scratch_types=[
            pltpu.VMEM((2, ISUP, CH), jnp.int32),
            pltpu.VMEM((2, ISUP, CH), jnp.int32),
            pltpu.VMEM((NBUF, CH, 48), jnp.int32),
            pltpu.VMEM((NBUF, CH, 32), jnp.int32),
            pltpu.VMEM((NBUF, CH, 2 * H), jnp.int32),
            pltpu.VMEM((SBUF, CH, TW), jnp.float32),
            pltpu.VMEM_SHARED((N_ACC, TW), jnp.float32),
            pltpu.SemaphoreType.DMA((NBUF,)),
            pltpu.SemaphoreType.DMA((SBUF,)),
            pltpu.SemaphoreType.DMA,
        ],
    )(_sc_edge_body)


def _sc_edge_body(ta_hbm, tb_hbm, er_hbm, src_hbm, dst_hbm, zero_hbm, out_hbm,
                  idx_s, idx_d, rows_a, rows_b, errs, sbuf, acc, gsem, ssem,
                  isem):
    c = lax.axis_index("c")
    s = lax.axis_index("s")
    pltpu.sync_copy(zero_hbm, acc.at[pl.ds(s * ROWS_PER_TILE, ROWS_PER_TILE)])
    plsc.subcore_barrier()

    w0 = (c * N_SUB + s) * CHUNKS

    def load_idx(sup, slot):
        pltpu.async_copy(src_hbm.at[pl.ds(w0 + sup * ISUP, ISUP)],
                         idx_s.at[slot], isem)
        pltpu.async_copy(dst_hbm.at[pl.ds(w0 + sup * ISUP, ISUP)],
                         idx_d.at[slot], isem)

    def wait_idx(slot):
        pltpu.make_async_copy(src_hbm.at[pl.ds(w0, ISUP)], idx_s.at[slot],
                              isem).wait()
        pltpu.make_async_copy(dst_hbm.at[pl.ds(w0, ISUP)], idx_d.at[slot],
                              isem).wait()

    def start_gather(slot, row, b):
        pltpu.async_copy(ta_hbm.at[idx_s.at[slot, row]], rows_a.at[b],
                         gsem.at[b])
        pltpu.async_copy(tb_hbm.at[idx_s.at[slot, row]], rows_b.at[b],
                         gsem.at[b])
        pltpu.async_copy(er_hbm.at[idx_d.at[slot, row]], errs.at[b],
                         gsem.at[b])

    def wait_gather(b):
        pltpu.make_async_copy(ta_hbm.at[idx_s.at[0, 0]], rows_a.at[b],
                              gsem.at[b]).wait()
        pltpu.make_async_copy(tb_hbm.at[idx_s.at[0, 0]], rows_b.at[b],
                              gsem.at[b]).wait()
        pltpu.make_async_copy(er_hbm.at[idx_d.at[0, 0]], errs.at[b],
                              gsem.at[b]).wait()

    def start_scatter(slot, row, p):
        pltpu.async_copy(sbuf.at[p], acc.at[idx_d.at[slot, row]], ssem.at[p],
                         add=True)

    def wait_scatter(p):
        pltpu.make_async_copy(sbuf.at[p], acc.at[idx_d.at[0, 0]],
                              ssem.at[p]).wait()

    def unpack2(vi):
        va = lax.bitcast_convert_type(lax.shift_left(vi, 16), jnp.float32)
        vb = lax.bitcast_convert_type(
            jnp.bitwise_and(vi, jnp.int32(-65536)), jnp.float32)
        return va, vb

    def compute(b, p):
        @plsc.parallel_loop(0, CH, unroll=4)
        def edge_body(e):
            elr, _ = unpack2(rows_b[b, e, pl.ds(16, 16)])
            erd, _ = unpack2(errs[b, e, :])
            ev = elr + erd
            ev = jnp.maximum(ev, 0.2 * ev)
            wv = jnp.exp(ev)
            sbuf[p, e, pl.ds(D, 16)] = wv
            for k in range(4):
                if k < 3:
                    ha, hb = unpack2(rows_a[b, e, pl.ds(16 * k, 16)])
                else:
                    ha, hb = unpack2(rows_b[b, e, pl.ds(0, 16)])
                ba = lax.broadcast(wv[2 * k], (16,))
                bb = lax.broadcast(wv[2 * k + 1], (16,))
                sbuf[p, e, pl.ds(F * 2 * k, F)] = ha * ba
                sbuf[p, e, pl.ds(F * (2 * k + 1), F)] = hb * bb

    pltpu.sync_copy(src_hbm.at[pl.ds(w0, ISUP)], idx_s.at[0])
    pltpu.sync_copy(dst_hbm.at[pl.ds(w0, ISUP)], idx_d.at[0])
    start_gather(0, 0, 0)
    start_gather(0, 1, 1)

    def chunk_body(i, carry):
        j = i // ISUP
        t = i - j * ISUP
        cs = j % 2
        ns = 1 - cs
        b = i % NBUF
        b2 = (i + 2) % NBUF
        p = i % SBUF
        more_sups = j < NSUP - 1

        @pl.when(jnp.logical_and(t == 1, more_sups))
        def _():
            load_idx(j + 1, ns)

        wait_gather(b)

        @pl.when(i > 1)
        def _():
            wait_scatter(p)

        compute(b, p)
        start_scatter(cs, t, p)

        @pl.when(jnp.logical_and(t == ISUP - 2, more_sups))
        def _():
            wait_idx(ns)

        tp = t + 2
        wrap = tp >= ISUP
        g_slot = jnp.where(wrap, ns, cs)
        g_row = jnp.where(wrap, tp - ISUP, tp)

        @pl.when(jnp.logical_or(jnp.logical_not(wrap), more_sups))
        def _():
            start_gather(g_slot, g_row, b2)

        return carry

    lax.fori_loop(0, CHUNKS, chunk_body, 0)
    wait_scatter((CHUNKS - 2) % SBUF)
    wait_scatter((CHUNKS - 1) % SBUF)
    plsc.subcore_barrier()
    pltpu.sync_copy(acc.at[pl.ds(s * ROWS_PER_TILE, ROWS_PER_TILE)],
                    out_hbm.at[c, pl.ds(s * ROWS_PER_TILE, ROWS_PER_TILE)])


def _block_diag(a):
    eye = jnp.eye(H, dtype=jnp.float32)
    return (a[:, :, None] * eye[:, None, :]).reshape(D, H)


def _tables_weights(al, ar):
    alr = jnp.concatenate([_block_diag(al), _block_diag(ar)], axis=1)
    i_alr = jnp.concatenate([jnp.eye(D, dtype=jnp.float32), alr], axis=1)
    g = i_alr @ jnp.asarray(_P)
    arr = jnp.concatenate([_block_diag(ar), _block_diag(ar)], axis=1)
    ger = arr @ jnp.asarray(_Q)
    return g, ger


def kernel(n_feat, edge_index, e_feat, W1, al1, ar1, b1, W2, al2, ar2, b2):
    del e_feat
    ei = edge_index.astype(jnp.int32)
    pad_e = E_PAD - E_RAW
    src = jnp.concatenate([ei[0], jnp.zeros((pad_e,), jnp.int32)])
    src = src.reshape(E_PAD // CH, CH)
    dst = jnp.concatenate([ei[1], jnp.full((pad_e,), N, jnp.int32)])
    dst = dst.reshape(E_PAD // CH, CH)
    zero_blk = jnp.zeros((ROWS_PER_TILE, TW), jnp.float32)
    rep = jnp.repeat(jnp.eye(H, dtype=jnp.float32), F, axis=1)

    g1, ger1 = _tables_weights(al1, ar1)
    g2, ger2 = _tables_weights(al2, ar2)

    def as_i32(t):
        return lax.bitcast_convert_type(
            t.reshape(t.shape[0], t.shape[1] // 2, 2), jnp.int32)

    x = jnp.pad(n_feat, ((0, N_ACC - N), (0, 0)))
    t1, er1 = _embed(x, W1, g1, ger1, rows_blk=2528)
    sc_edge = _make_sc_edge()
    t1i = as_i32(t1)
    acc1 = sc_edge(t1i[:, :48], t1i[:, 48:], as_i32(er1), src, dst, zero_blk)
    t2, er2 = _norm_embed(acc1, rep, b1.reshape(1, D), W2, g2, ger2,
                          rows_blk=2528)
    t2i = as_i32(t2)
    acc2 = sc_edge(t2i[:, :48], t2i[:, 48:], as_i32(er2), src, dst, zero_blk)
    return _final(acc2, rep, b2.reshape(1, D), rows_blk=2000)


# speedup vs baseline: 2.2855x; 1.2629x over previous
import functools

import jax
import jax.numpy as jnp
import numpy as np
from jax import lax
from jax.experimental import pallas as pl
from jax.experimental.pallas import tpu as pltpu
from jax.experimental.pallas import tpu_sc as plsc

N = 10000
D = 128
H = 8
F = 16
TW = D + 2 * H
TM = 160
N_CORES = 2
N_SUB = 16
N_WORK = N_CORES * N_SUB
CH = 64
E_RAW = 320000
NBUF = 3
SBUF = 2
ISUP = 9
NSUP = 18
CHUNKS = ISUP * NSUP
E_PAD = N_WORK * CH * CHUNKS
EPW = CH * CHUNKS
N_ACC = 10112
ROWS_PER_TILE = N_ACC // N_SUB

_P = np.zeros((TW, TM), np.float32)
for _m in range(TM):
    _k, _r = divmod(_m, 32)
    if _k < 4:
        _P[32 * _k + 16 * (_r % 2) + _r // 2, _m] = 1.0
    else:
        _P[D + _r // 2, _m] = 1.0
_Q = np.zeros((2 * H, 4 * H), np.float32)
for _j in range(2 * H):
    _Q[_j, 2 * _j] = 1.0
    _Q[_j, 2 * _j + 1] = 1.0


def _embed_body(x_ref, w_ref, g_ref, ger_ref, t_ref, er_ref):
    h = jnp.dot(x_ref[...], w_ref[...], preferred_element_type=jnp.float32)
    t_ref[...] = jnp.dot(h, g_ref[...],
                         preferred_element_type=jnp.float32).astype(jnp.bfloat16)
    er_ref[...] = jnp.dot(h, ger_ref[...],
                          preferred_element_type=jnp.float32).astype(jnp.bfloat16)


def _embed(x, w, g, ger, rows_blk):
    n = x.shape[0]
    return pl.pallas_call(
        _embed_body,
        grid=(n // rows_blk,),
        in_specs=[
            pl.BlockSpec((rows_blk, D), lambda i: (i, 0)),
            pl.BlockSpec((D, D), lambda i: (0, 0)),
            pl.BlockSpec((D, TM), lambda i: (0, 0)),
            pl.BlockSpec((D, 4 * H), lambda i: (0, 0)),
        ],
        out_specs=[
            pl.BlockSpec((rows_blk, TM), lambda i: (i, 0)),
            pl.BlockSpec((rows_blk, 4 * H), lambda i: (i, 0)),
        ],
        out_shape=[
            jax.ShapeDtypeStruct((n, TM), jnp.bfloat16),
            jax.ShapeDtypeStruct((n, 4 * H), jnp.bfloat16),
        ],
    )(x, w, g, ger)


def _norm_embed_body(acc_ref, rep_ref, b_ref, w_ref, g_ref, ger_ref,
                     t_ref, er_ref):
    s = acc_ref[0] + acc_ref[1]
    den = jnp.dot(s[:, D:D + H], rep_ref[...], preferred_element_type=jnp.float32)
    x1 = s[:, :D] / (den + 1e-9) + b_ref[...]
    x1 = jnp.maximum(x1, 0.01 * x1)
    h = jnp.dot(x1, w_ref[...], preferred_element_type=jnp.float32)
    t_ref[...] = jnp.dot(h, g_ref[...],
                         preferred_element_type=jnp.float32).astype(jnp.bfloat16)
    er_ref[...] = jnp.dot(h, ger_ref[...],
                          preferred_element_type=jnp.float32).astype(jnp.bfloat16)


def _norm_embed(acc, rep, b, w, g, ger, rows_blk):
    n = acc.shape[1]
    return pl.pallas_call(
        _norm_embed_body,
        grid=(n // rows_blk,),
        in_specs=[
            pl.BlockSpec((2, rows_blk, TW), lambda i: (0, i, 0)),
            pl.BlockSpec((H, D), lambda i: (0, 0)),
            pl.BlockSpec((1, D), lambda i: (0, 0)),
            pl.BlockSpec((D, D), lambda i: (0, 0)),
            pl.BlockSpec((D, TM), lambda i: (0, 0)),
            pl.BlockSpec((D, 4 * H), lambda i: (0, 0)),
        ],
        out_specs=[
            pl.BlockSpec((rows_blk, TM), lambda i: (i, 0)),
            pl.BlockSpec((rows_blk, 4 * H), lambda i: (i, 0)),
        ],
        out_shape=[
            jax.ShapeDtypeStruct((n, TM), jnp.bfloat16),
            jax.ShapeDtypeStruct((n, 4 * H), jnp.bfloat16),
        ],
    )(acc, rep, b, w, g, ger)


def _final_body(acc_ref, rep_ref, b_ref, o_ref):
    s = acc_ref[0] + acc_ref[1]
    den = jnp.dot(s[:, D:D + H], rep_ref[...], preferred_element_type=jnp.float32)
    o_ref[...] = s[:, :D] / (den + 1e-9) + b_ref[...]


def _final(acc, rep, b, rows_blk):
    return pl.pallas_call(
        _final_body,
        grid=(N // rows_blk,),
        in_specs=[
            pl.BlockSpec((2, rows_blk, TW), lambda i: (0, i, 0)),
            pl.BlockSpec((H, D), lambda i: (0, 0)),
            pl.BlockSpec((1, D), lambda i: (0, 0)),
        ],
        out_specs=pl.BlockSpec((rows_blk, D), lambda i: (i, 0)),
        out_shape=jax.ShapeDtypeStruct((N, D), jnp.float32),
    )(acc, rep, b)


@functools.cache
def _make_sc_edge():
    mesh = plsc.VectorSubcoreMesh(core_axis_name="c", subcore_axis_name="s")
    return functools.partial(
        pl.kernel,
        mesh=mesh,
        compiler_params=pltpu.CompilerParams(use_tc_tiling_on_sc=False),
        out_type=jax.ShapeDtypeStruct((N_CORES, N_ACC, TW), jnp.float32),
        scratch_types=[
            pltpu.VMEM((2, ISUP, CH), jnp.int32),
            pltpu.VMEM((2, ISUP, CH), jnp.int32),
            pltpu.VMEM((NBUF, 5, CH, 2 * H), jnp.int32),
            pltpu.VMEM((NBUF, CH, 2 * H), jnp.int32),
            pltpu.VMEM((SBUF, CH, TW), jnp.float32),
            pltpu.VMEM_SHARED((N_ACC, TW), jnp.float32),
            pltpu.SemaphoreType.DMA((NBUF,)),
            pltpu.SemaphoreType.DMA((SBUF,)),
            pltpu.SemaphoreType.DMA,
        ],
    )(_sc_edge_body)


def _sc_edge_body(t0_hbm, t1_hbm, t2_hbm, t3_hbm, t4_hbm, er_hbm, src_hbm,
                  dst_hbm, zero_hbm, out_hbm, idx_s, idx_d, rows, errs, sbuf,
                  acc, gsem, ssem, isem):
    c = lax.axis_index("c")
    s = lax.axis_index("s")
    pltpu.sync_copy(zero_hbm, acc.at[pl.ds(s * ROWS_PER_TILE, ROWS_PER_TILE)])
    plsc.subcore_barrier()

    w0 = (c * N_SUB + s) * CHUNKS

    def load_idx(sup, slot):
        pltpu.async_copy(src_hbm.at[pl.ds(w0 + sup * ISUP, ISUP)],
                         idx_s.at[slot], isem)
        pltpu.async_copy(dst_hbm.at[pl.ds(w0 + sup * ISUP, ISUP)],
                         idx_d.at[slot], isem)

    def wait_idx(slot):
        pltpu.make_async_copy(src_hbm.at[pl.ds(w0, ISUP)], idx_s.at[slot],
                              isem).wait()
        pltpu.make_async_copy(dst_hbm.at[pl.ds(w0, ISUP)], idx_d.at[slot],
                              isem).wait()

    t_tabs = (t0_hbm, t1_hbm, t2_hbm, t3_hbm, t4_hbm)

    def start_gather(slot, row, b):
        for g in range(5):
            pltpu.async_copy(t_tabs[g].at[idx_s.at[slot, row]], rows.at[b, g],
                             gsem.at[b])
        pltpu.async_copy(er_hbm.at[idx_d.at[slot, row]], errs.at[b],
                         gsem.at[b])

    def wait_gather(b):
        for g in range(5):
            pltpu.make_async_copy(t_tabs[g].at[idx_s.at[0, 0]], rows.at[b, g],
                                  gsem.at[b]).wait()
        pltpu.make_async_copy(er_hbm.at[idx_d.at[0, 0]], errs.at[b],
                              gsem.at[b]).wait()

    def start_scatter(slot, row, p):
        pltpu.async_copy(sbuf.at[p], acc.at[idx_d.at[slot, row]], ssem.at[p],
                         add=True)

    def wait_scatter(p):
        pltpu.make_async_copy(sbuf.at[p], acc.at[idx_d.at[0, 0]],
                              ssem.at[p]).wait()

    def unpack2(vi):
        va = lax.bitcast_convert_type(lax.shift_left(vi, 16), jnp.float32)
        vb = lax.bitcast_convert_type(
            jnp.bitwise_and(vi, jnp.int32(-65536)), jnp.float32)
        return va, vb

    def compute(b, p):
        @plsc.parallel_loop(0, CH, unroll=4)
        def edge_body(e):
            elr, _ = unpack2(rows[b, 4, e, :])
            erd, _ = unpack2(errs[b, e, :])
            ev = elr + erd
            ev = jnp.maximum(ev, 0.2 * ev)
            wv = jnp.exp(ev)
            sbuf[p, e, pl.ds(D, 16)] = wv
            for k in range(4):
                ha, hb = unpack2(rows[b, k, e, :])
                ba = lax.broadcast(wv[2 * k], (16,))
                bb = lax.broadcast(wv[2 * k + 1], (16,))
                sbuf[p, e, pl.ds(F * 2 * k, F)] = ha * ba
                sbuf[p, e, pl.ds(F * (2 * k + 1), F)] = hb * bb

    pltpu.sync_copy(src_hbm.at[pl.ds(w0, ISUP)], idx_s.at[0])
    pltpu.sync_copy(dst_hbm.at[pl.ds(w0, ISUP)], idx_d.at[0])
    start_gather(0, 0, 0)
    start_gather(0, 1, 1)

    def chunk_body(i, carry):
        j = i // ISUP
        t = i - j * ISUP
        cs = j % 2
        ns = 1 - cs
        b = i % NBUF
        b2 = (i + 2) % NBUF
        p = i % SBUF
        more_sups = j < NSUP - 1

        @pl.when(jnp.logical_and(t == 1, more_sups))
        def _():
            load_idx(j + 1, ns)

        wait_gather(b)

        @pl.when(i > 1)
        def _():
            wait_scatter(p)

        compute(b, p)
        start_scatter(cs, t, p)

        @pl.when(jnp.logical_and(t == ISUP - 2, more_sups))
        def _():
            wait_idx(ns)

        tp = t + 2
        wrap = tp >= ISUP
        g_slot = jnp.where(wrap, ns, cs)
        g_row = jnp.where(wrap, tp - ISUP, tp)

        @pl.when(jnp.logical_or(jnp.logical_not(wrap), more_sups))
        def _():
            start_gather(g_slot, g_row, b2)

        return carry

    lax.fori_loop(0, CHUNKS, chunk_body, 0)
    wait_scatter((CHUNKS - 2) % SBUF)
    wait_scatter((CHUNKS - 1) % SBUF)
    plsc.subcore_barrier()
    pltpu.sync_copy(acc.at[pl.ds(s * ROWS_PER_TILE, ROWS_PER_TILE)],
                    out_hbm.at[c, pl.ds(s * ROWS_PER_TILE, ROWS_PER_TILE)])


def _block_diag(a):
    eye = jnp.eye(H, dtype=jnp.float32)
    return (a[:, :, None] * eye[:, None, :]).reshape(D, H)


def _tables_weights(al, ar):
    alr = jnp.concatenate([_block_diag(al), _block_diag(ar)], axis=1)
    i_alr = jnp.concatenate([jnp.eye(D, dtype=jnp.float32), alr], axis=1)
    g = i_alr @ jnp.asarray(_P)
    arr = jnp.concatenate([_block_diag(ar), _block_diag(ar)], axis=1)
    ger = arr @ jnp.asarray(_Q)
    return g, ger


def kernel(n_feat, edge_index, e_feat, W1, al1, ar1, b1, W2, al2, ar2, b2):
    del e_feat
    ei = edge_index.astype(jnp.int32)
    pad_e = E_PAD - E_RAW
    src = jnp.concatenate([ei[0], jnp.zeros((pad_e,), jnp.int32)])
    src = src.reshape(E_PAD // CH, CH)
    dst = jnp.concatenate([ei[1], jnp.full((pad_e,), N, jnp.int32)])
    dst = dst.reshape(E_PAD // CH, CH)
    zero_blk = jnp.zeros((ROWS_PER_TILE, TW), jnp.float32)
    rep = jnp.repeat(jnp.eye(H, dtype=jnp.float32), F, axis=1)

    g1, ger1 = _tables_weights(al1, ar1)
    g2, ger2 = _tables_weights(al2, ar2)

    def as_i32(t):
        return lax.bitcast_convert_type(
            t.reshape(t.shape[0], t.shape[1] // 2, 2), jnp.int32)

    x = jnp.pad(n_feat, ((0, N_ACC - N), (0, 0)))
    t1, er1 = _embed(x, W1, g1, ger1, rows_blk=2528)
    sc_edge = _make_sc_edge()
    t1i = as_i32(t1)
    acc1 = sc_edge(*[t1i[:, 16 * g:16 * g + 16] for g in range(5)],
                   as_i32(er1), src, dst, zero_blk)
    t2, er2 = _norm_embed(acc1, rep, b1.reshape(1, D), W2, g2, ger2,
                          rows_blk=2528)
    t2i = as_i32(t2)
    acc2 = sc_edge(*[t2i[:, 16 * g:16 * g + 16] for g in range(5)],
                   as_i32(er2), src, dst, zero_blk)
    return _final(acc2, rep, b2.reshape(1, D), rows_blk=2000)


# speedup vs baseline: 2.2984x; 1.0056x over previous
import functools

import jax
import jax.numpy as jnp
import numpy as np
from jax import lax
from jax.experimental import pallas as pl
from jax.experimental.pallas import tpu as pltpu
from jax.experimental.pallas import tpu_sc as plsc

N = 10000
D = 128
H = 8
F = 16
TW = D + 2 * H
TM = 160
N_CORES = 2
N_SUB = 16
N_WORK = N_CORES * N_SUB
CH = 128
SCH = 32
NSC = CH // SCH
E_RAW = 320000
NBUF = 2
SBUF = 2
ISUP = 9
NSUP = 9
CHUNKS = ISUP * NSUP
E_PAD = N_WORK * CH * CHUNKS
EPW = CH * CHUNKS
N_ACC = 10112
ROWS_PER_TILE = N_ACC // N_SUB

_P = np.zeros((TW, TM), np.float32)
for _m in range(TM):
    _k, _r = divmod(_m, 32)
    if _k < 4:
        _P[32 * _k + 16 * (_r % 2) + _r // 2, _m] = 1.0
    else:
        _P[D + _r // 2, _m] = 1.0
_Q = np.zeros((2 * H, 4 * H), np.float32)
for _j in range(2 * H):
    _Q[_j, 2 * _j] = 1.0
    _Q[_j, 2 * _j + 1] = 1.0


def _embed_body(x_ref, w_ref, g_ref, ger_ref, t_ref, er_ref):
    h = jnp.dot(x_ref[...], w_ref[...], preferred_element_type=jnp.float32)
    t_ref[...] = jnp.dot(h, g_ref[...],
                         preferred_element_type=jnp.float32).astype(jnp.bfloat16)
    er_ref[...] = jnp.dot(h, ger_ref[...],
                          preferred_element_type=jnp.float32).astype(jnp.bfloat16)


def _embed(x, w, g, ger, rows_blk):
    n = x.shape[0]
    return pl.pallas_call(
        _embed_body,
        grid=(n // rows_blk,),
        in_specs=[
            pl.BlockSpec((rows_blk, D), lambda i: (i, 0)),
            pl.BlockSpec((D, D), lambda i: (0, 0)),
            pl.BlockSpec((D, TM), lambda i: (0, 0)),
            pl.BlockSpec((D, 4 * H), lambda i: (0, 0)),
        ],
        out_specs=[
            pl.BlockSpec((rows_blk, TM), lambda i: (i, 0)),
            pl.BlockSpec((rows_blk, 4 * H), lambda i: (i, 0)),
        ],
        out_shape=[
            jax.ShapeDtypeStruct((n, TM), jnp.bfloat16),
            jax.ShapeDtypeStruct((n, 4 * H), jnp.bfloat16),
        ],
    )(x, w, g, ger)


def _norm_embed_body(acc_ref, rep_ref, b_ref, w_ref, g_ref, ger_ref,
                     t_ref, er_ref):
    s = acc_ref[0] + acc_ref[1]
    den = jnp.dot(s[:, D:D + H], rep_ref[...], preferred_element_type=jnp.float32)
    x1 = s[:, :D] / (den + 1e-9) + b_ref[...]
    x1 = jnp.maximum(x1, 0.01 * x1)
    h = jnp.dot(x1, w_ref[...], preferred_element_type=jnp.float32)
    t_ref[...] = jnp.dot(h, g_ref[...],
                         preferred_element_type=jnp.float32).astype(jnp.bfloat16)
    er_ref[...] = jnp.dot(h, ger_ref[...],
                          preferred_element_type=jnp.float32).astype(jnp.bfloat16)


def _norm_embed(acc, rep, b, w, g, ger, rows_blk):
    n = acc.shape[1]
    return pl.pallas_call(
        _norm_embed_body,
        grid=(n // rows_blk,),
        in_specs=[
            pl.BlockSpec((2, rows_blk, TW), lambda i: (0, i, 0)),
            pl.BlockSpec((H, D), lambda i: (0, 0)),
            pl.BlockSpec((1, D), lambda i: (0, 0)),
            pl.BlockSpec((D, D), lambda i: (0, 0)),
            pl.BlockSpec((D, TM), lambda i: (0, 0)),
            pl.BlockSpec((D, 4 * H), lambda i: (0, 0)),
        ],
        out_specs=[
            pl.BlockSpec((rows_blk, TM), lambda i: (i, 0)),
            pl.BlockSpec((rows_blk, 4 * H), lambda i: (i, 0)),
        ],
        out_shape=[
            jax.ShapeDtypeStruct((n, TM), jnp.bfloat16),
            jax.ShapeDtypeStruct((n, 4 * H), jnp.bfloat16),
        ],
    )(acc, rep, b, w, g, ger)


def _final_body(acc_ref, rep_ref, b_ref, o_ref):
    s = acc_ref[0] + acc_ref[1]
    den = jnp.dot(s[:, D:D + H], rep_ref[...], preferred_element_type=jnp.float32)
    o_ref[...] = s[:, :D] / (den + 1e-9) + b_ref[...]


def _final(acc, rep, b, rows_blk):
    return pl.pallas_call(
        _final_body,
        grid=(N // rows_blk,),
        in_specs=[
            pl.BlockSpec((2, rows_blk, TW), lambda i: (0, i, 0)),
            pl.BlockSpec((H, D), lambda i: (0, 0)),
            pl.BlockSpec((1, D), lambda i: (0, 0)),
        ],
        out_specs=pl.BlockSpec((rows_blk, D), lambda i: (i, 0)),
        out_shape=jax.ShapeDtypeStruct((N, D), jnp.float32),
    )(acc, rep, b)


@functools.cache
def _make_sc_edge():
    mesh = plsc.VectorSubcoreMesh(core_axis_name="c", subcore_axis_name="s")
    return functools.partial(
        pl.kernel,
        mesh=mesh,
        compiler_params=pltpu.CompilerParams(use_tc_tiling_on_sc=False),
        out_type=jax.ShapeDtypeStruct((N_CORES, N_ACC, TW), jnp.float32),
        scratch_types=[
            pltpu.VMEM((2, ISUP, CH), jnp.int32),
            pltpu.VMEM((2, ISUP, NSC, SCH), jnp.int32),
            pltpu.VMEM((NBUF, 5, CH, 2 * H), jnp.int32),
            pltpu.VMEM((NBUF, CH, 2 * H), jnp.int32),
            pltpu.VMEM((SBUF, SCH, TW), jnp.float32),
            pltpu.VMEM_SHARED((N_ACC, TW), jnp.float32),
            pltpu.SemaphoreType.DMA((NBUF,)),
            pltpu.SemaphoreType.DMA((SBUF,)),
            pltpu.SemaphoreType.DMA,
        ],
    )(_sc_edge_body)


def _sc_edge_body(t0_hbm, t1_hbm, t2_hbm, t3_hbm, t4_hbm, er_hbm, src_hbm,
                  dst_hbm, zero_hbm, out_hbm, idx_s, idx_d, rows, errs, sbuf,
                  acc, gsem, ssem, isem):
    c = lax.axis_index("c")
    s = lax.axis_index("s")
    pltpu.sync_copy(zero_hbm, acc.at[pl.ds(s * ROWS_PER_TILE, ROWS_PER_TILE)])
    plsc.subcore_barrier()

    w0 = (c * N_SUB + s) * CHUNKS

    def load_idx(sup, slot):
        pltpu.async_copy(src_hbm.at[pl.ds(w0 + sup * ISUP, ISUP)],
                         idx_s.at[slot], isem)
        pltpu.async_copy(dst_hbm.at[pl.ds(w0 + sup * ISUP, ISUP)],
                         idx_d.at[slot], isem)

    def wait_idx(slot):
        pltpu.make_async_copy(src_hbm.at[pl.ds(w0, ISUP)], idx_s.at[slot],
                              isem).wait()
        pltpu.make_async_copy(dst_hbm.at[pl.ds(w0, ISUP)], idx_d.at[slot],
                              isem).wait()

    t_tabs = (t0_hbm, t1_hbm, t2_hbm, t3_hbm, t4_hbm)

    def start_gather(slot, row, b):
        for g in range(5):
            pltpu.async_copy(t_tabs[g].at[idx_s.at[slot, row]], rows.at[b, g],
                             gsem.at[b])
        for sub in range(NSC):
            pltpu.async_copy(er_hbm.at[idx_d.at[slot, row, sub]],
                             errs.at[b, pl.ds(sub * SCH, SCH)], gsem.at[b])

    def wait_gather(b):
        for g in range(5):
            pltpu.make_async_copy(t_tabs[g].at[idx_s.at[0, 0]], rows.at[b, g],
                                  gsem.at[b]).wait()
        for sub in range(NSC):
            pltpu.make_async_copy(er_hbm.at[idx_d.at[0, 0, 0]],
                                  errs.at[b, pl.ds(sub * SCH, SCH)],
                                  gsem.at[b]).wait()

    def start_scatter(slot, row, sub, p):
        pltpu.async_copy(sbuf.at[p], acc.at[idx_d.at[slot, row, sub]],
                         ssem.at[p], add=True)

    def wait_scatter(p):
        pltpu.make_async_copy(sbuf.at[p], acc.at[idx_d.at[0, 0, 0]],
                              ssem.at[p]).wait()

    def unpack2(vi):
        va = lax.bitcast_convert_type(lax.shift_left(vi, 16), jnp.float32)
        vb = lax.bitcast_convert_type(
            jnp.bitwise_and(vi, jnp.int32(-65536)), jnp.float32)
        return va, vb

    def compute(b, p, sub):
        @plsc.parallel_loop(sub * SCH, (sub + 1) * SCH, unroll=4)
        def edge_body(e):
            es = e - sub * SCH
            elr, _ = unpack2(rows[b, 4, e, :])
            erd, _ = unpack2(errs[b, e, :])
            ev = elr + erd
            ev = jnp.maximum(ev, 0.2 * ev)
            wv = jnp.exp(ev)
            sbuf[p, es, pl.ds(D, 16)] = wv
            for k in range(4):
                ha, hb = unpack2(rows[b, k, e, :])
                ba = lax.broadcast(wv[2 * k], (16,))
                bb = lax.broadcast(wv[2 * k + 1], (16,))
                sbuf[p, es, pl.ds(F * 2 * k, F)] = ha * ba
                sbuf[p, es, pl.ds(F * (2 * k + 1), F)] = hb * bb

    pltpu.sync_copy(src_hbm.at[pl.ds(w0, ISUP)], idx_s.at[0])
    pltpu.sync_copy(dst_hbm.at[pl.ds(w0, ISUP)], idx_d.at[0])
    start_gather(0, 0, 0)
    start_gather(0, 1, 1)

    def chunk_body(i, carry):
        j = i // ISUP
        t = i - j * ISUP
        cs = j % 2
        ns = 1 - cs
        b = i % NBUF
        b2 = (i + 2) % NBUF
        more_sups = j < NSUP - 1

        @pl.when(jnp.logical_and(t == 1, more_sups))
        def _():
            load_idx(j + 1, ns)

        wait_gather(b)

        for sub in range(NSC):
            ps = sub % SBUF
            if sub < SBUF:
                @pl.when(i > 0)
                def _(ps=ps):
                    wait_scatter(ps)
            else:
                wait_scatter(ps)
            compute(b, ps, sub)
            start_scatter(cs, t, sub, ps)

        @pl.when(jnp.logical_and(t == ISUP - 2, more_sups))
        def _():
            wait_idx(ns)

        tp = t + 2
        wrap = tp >= ISUP
        g_slot = jnp.where(wrap, ns, cs)
        g_row = jnp.where(wrap, tp - ISUP, tp)

        @pl.when(jnp.logical_or(jnp.logical_not(wrap), more_sups))
        def _():
            start_gather(g_slot, g_row, b2)

        return carry

    lax.fori_loop(0, CHUNKS, chunk_body, 0)
    wait_scatter(0)
    wait_scatter(1)
    plsc.subcore_barrier()
    pltpu.sync_copy(acc.at[pl.ds(s * ROWS_PER_TILE, ROWS_PER_TILE)],
                    out_hbm.at[c, pl.ds(s * ROWS_PER_TILE, ROWS_PER_TILE)])


def _block_diag(a):
    eye = jnp.eye(H, dtype=jnp.float32)
    return (a[:, :, None] * eye[:, None, :]).reshape(D, H)


def _tables_weights(al, ar):
    alr = jnp.concatenate([_block_diag(al), _block_diag(ar)], axis=1)
    i_alr = jnp.concatenate([jnp.eye(D, dtype=jnp.float32), alr], axis=1)
    g = i_alr @ jnp.asarray(_P)
    arr = jnp.concatenate([_block_diag(ar), _block_diag(ar)], axis=1)
    ger = arr @ jnp.asarray(_Q)
    return g, ger


def kernel(n_feat, edge_index, e_feat, W1, al1, ar1, b1, W2, al2, ar2, b2):
    del e_feat
    ei = edge_index.astype(jnp.int32)
    pad_e = E_PAD - E_RAW
    src = jnp.concatenate([ei[0], jnp.zeros((pad_e,), jnp.int32)])
    src = src.reshape(E_PAD // CH, CH)
    dst = jnp.concatenate([ei[1], jnp.full((pad_e,), N, jnp.int32)])
    dst = dst.reshape(E_PAD // CH, NSC, SCH)
    zero_blk = jnp.zeros((ROWS_PER_TILE, TW), jnp.float32)
    rep = jnp.repeat(jnp.eye(H, dtype=jnp.float32), F, axis=1)

    g1, ger1 = _tables_weights(al1, ar1)
    g2, ger2 = _tables_weights(al2, ar2)

    def as_i32(t):
        return lax.bitcast_convert_type(
            t.reshape(t.shape[0], t.shape[1] // 2, 2), jnp.int32)

    x = jnp.pad(n_feat, ((0, N_ACC - N), (0, 0)))
    t1, er1 = _embed(x, W1, g1, ger1, rows_blk=2528)
    sc_edge = _make_sc_edge()
    t1i = as_i32(t1)
    acc1 = sc_edge(*[t1i[:, 16 * g:16 * g + 16] for g in range(5)],
                   as_i32(er1), src, dst, zero_blk)
    t2, er2 = _norm_embed(acc1, rep, b1.reshape(1, D), W2, g2, ger2,
                          rows_blk=2528)
    t2i = as_i32(t2)
    acc2 = sc_edge(*[t2i[:, 16 * g:16 * g + 16] for g in range(5)],
                   as_i32(er2), src, dst, zero_blk)
    return _final(acc2, rep, b2.reshape(1, D), rows_blk=2000)


# speedup vs baseline: 2.3289x; 1.0133x over previous
import functools

import jax
import jax.numpy as jnp
import numpy as np
from jax import lax
from jax.experimental import pallas as pl
from jax.experimental.pallas import tpu as pltpu
from jax.experimental.pallas import tpu_sc as plsc

N = 10000
D = 128
H = 8
F = 16
TW = D + 2 * H
TM = 160
N_CORES = 2
N_SUB = 16
N_WORK = N_CORES * N_SUB
CH = 128
SCH = 32
NSC = CH // SCH
E_RAW = 320000
NBUF = 2
SBUF = 2
ISUP = 9
NSUP = 9
CHUNKS = ISUP * NSUP
E_PAD = N_WORK * CH * CHUNKS
EPW = CH * CHUNKS
N_ACC = 10112
ROWS_PER_TILE = N_ACC // N_SUB

_P = np.zeros((TW, TM), np.float32)
for _m in range(TM):
    _k, _r = divmod(_m, 32)
    if _k < 4:
        _P[32 * _k + 16 * (_r % 2) + _r // 2, _m] = 1.0
    else:
        _P[D + _r // 2, _m] = 1.0
_Q = np.zeros((2 * H, 4 * H), np.float32)
for _j in range(2 * H):
    _Q[_j, 2 * _j] = 1.0
    _Q[_j, 2 * _j + 1] = 1.0


def _embed_body(x_ref, w_ref, g_ref, ger_ref, t_ref, er_ref):
    h = jnp.dot(x_ref[...], w_ref[...], preferred_element_type=jnp.float32)
    t_ref[...] = jnp.dot(h, g_ref[...],
                         preferred_element_type=jnp.float32).astype(jnp.bfloat16)
    er_ref[...] = jnp.dot(h, ger_ref[...],
                          preferred_element_type=jnp.float32).astype(jnp.bfloat16)


def _embed(x, w, g, ger, rows_blk):
    n = x.shape[0]
    return pl.pallas_call(
        _embed_body,
        grid=(n // rows_blk,),
        in_specs=[
            pl.BlockSpec((rows_blk, D), lambda i: (i, 0)),
            pl.BlockSpec((D, D), lambda i: (0, 0)),
            pl.BlockSpec((D, TM), lambda i: (0, 0)),
            pl.BlockSpec((D, 4 * H), lambda i: (0, 0)),
        ],
        out_specs=[
            pl.BlockSpec((rows_blk, TM), lambda i: (i, 0)),
            pl.BlockSpec((rows_blk, 4 * H), lambda i: (i, 0)),
        ],
        out_shape=[
            jax.ShapeDtypeStruct((n, TM), jnp.bfloat16),
            jax.ShapeDtypeStruct((n, 4 * H), jnp.bfloat16),
        ],
    )(x, w, g, ger)


def _norm_embed_body(acc_ref, rep_ref, b_ref, w_ref, g_ref, ger_ref,
                     t_ref, er_ref):
    s = acc_ref[0] + acc_ref[1]
    den = jnp.dot(s[:, D:D + H], rep_ref[...], preferred_element_type=jnp.float32)
    x1 = s[:, :D] / (den + 1e-9) + b_ref[...]
    x1 = jnp.maximum(x1, 0.01 * x1)
    h = jnp.dot(x1, w_ref[...], preferred_element_type=jnp.float32)
    t_ref[...] = jnp.dot(h, g_ref[...],
                         preferred_element_type=jnp.float32).astype(jnp.bfloat16)
    er_ref[...] = jnp.dot(h, ger_ref[...],
                          preferred_element_type=jnp.float32).astype(jnp.bfloat16)


def _norm_embed(acc, rep, b, w, g, ger, rows_blk):
    n = acc.shape[1]
    return pl.pallas_call(
        _norm_embed_body,
        grid=(n // rows_blk,),
        in_specs=[
            pl.BlockSpec((2, rows_blk, TW), lambda i: (0, i, 0)),
            pl.BlockSpec((H, D), lambda i: (0, 0)),
            pl.BlockSpec((1, D), lambda i: (0, 0)),
            pl.BlockSpec((D, D), lambda i: (0, 0)),
            pl.BlockSpec((D, TM), lambda i: (0, 0)),
            pl.BlockSpec((D, 4 * H), lambda i: (0, 0)),
        ],
        out_specs=[
            pl.BlockSpec((rows_blk, TM), lambda i: (i, 0)),
            pl.BlockSpec((rows_blk, 4 * H), lambda i: (i, 0)),
        ],
        out_shape=[
            jax.ShapeDtypeStruct((n, TM), jnp.bfloat16),
            jax.ShapeDtypeStruct((n, 4 * H), jnp.bfloat16),
        ],
    )(acc, rep, b, w, g, ger)


def _final_body(acc_ref, rep_ref, b_ref, o_ref):
    s = acc_ref[0] + acc_ref[1]
    den = jnp.dot(s[:, D:D + H], rep_ref[...], preferred_element_type=jnp.float32)
    o_ref[...] = s[:, :D] / (den + 1e-9) + b_ref[...]


def _final(acc, rep, b, rows_blk):
    return pl.pallas_call(
        _final_body,
        grid=(N // rows_blk,),
        in_specs=[
            pl.BlockSpec((2, rows_blk, TW), lambda i: (0, i, 0)),
            pl.BlockSpec((H, D), lambda i: (0, 0)),
            pl.BlockSpec((1, D), lambda i: (0, 0)),
        ],
        out_specs=pl.BlockSpec((rows_blk, D), lambda i: (i, 0)),
        out_shape=jax.ShapeDtypeStruct((N, D), jnp.float32),
    )(acc, rep, b)


@functools.cache
def _make_sc_edge():
    mesh = plsc.VectorSubcoreMesh(core_axis_name="c", subcore_axis_name="s")
    return functools.partial(
        pl.kernel,
        mesh=mesh,
        compiler_params=pltpu.CompilerParams(use_tc_tiling_on_sc=False),
        out_type=jax.ShapeDtypeStruct((N_CORES, N_ACC, TW), jnp.float32),
        scratch_types=[
            pltpu.VMEM((2, ISUP, CH), jnp.int32),
            pltpu.VMEM((2, ISUP, NSC, SCH), jnp.int32),
            pltpu.VMEM((NBUF, 5, CH, 2 * H), jnp.int32),
            pltpu.VMEM((NBUF, CH, 2 * H), jnp.int32),
            pltpu.VMEM((SBUF, SCH, TW), jnp.float32),
            pltpu.VMEM_SHARED((N_ACC, TW), jnp.float32),
            pltpu.SemaphoreType.DMA((NBUF,)),
            pltpu.SemaphoreType.DMA((SBUF,)),
            pltpu.SemaphoreType.DMA,
        ],
    )(_sc_edge_body)


def _sc_edge_body(t0_hbm, t1_hbm, t2_hbm, t3_hbm, t4_hbm, er_hbm, src_hbm,
                  dst_hbm, zero_hbm, out_hbm, idx_s, idx_d, rows, errs, sbuf,
                  acc, gsem, ssem, isem):
    c = lax.axis_index("c")
    s = lax.axis_index("s")
    pltpu.sync_copy(zero_hbm, acc.at[pl.ds(s * ROWS_PER_TILE, ROWS_PER_TILE)])
    plsc.subcore_barrier()

    w0 = (c * N_SUB + s) * CHUNKS

    def load_idx(sup, slot):
        pltpu.async_copy(src_hbm.at[pl.ds(w0 + sup * ISUP, ISUP)],
                         idx_s.at[slot], isem)
        pltpu.async_copy(dst_hbm.at[pl.ds(w0 + sup * ISUP, ISUP)],
                         idx_d.at[slot], isem)

    def wait_idx(slot):
        pltpu.make_async_copy(src_hbm.at[pl.ds(w0, ISUP)], idx_s.at[slot],
                              isem).wait()
        pltpu.make_async_copy(dst_hbm.at[pl.ds(w0, ISUP)], idx_d.at[slot],
                              isem).wait()

    t_tabs = (t0_hbm, t1_hbm, t2_hbm, t3_hbm, t4_hbm)

    def start_gather(slot, row, b):
        for g in range(5):
            for h in range(2):
                pltpu.async_copy(
                    t_tabs[g].at[idx_s.at[slot, row, pl.ds(h * 64, 64)]],
                    rows.at[b, g, pl.ds(h * 64, 64)], gsem.at[b])
        for sub in range(NSC):
            pltpu.async_copy(er_hbm.at[idx_d.at[slot, row, sub]],
                             errs.at[b, pl.ds(sub * SCH, SCH)], gsem.at[b])

    def wait_gather(b):
        for g in range(5):
            for h in range(2):
                pltpu.make_async_copy(
                    t_tabs[g].at[idx_s.at[0, 0, pl.ds(h * 64, 64)]],
                    rows.at[b, g, pl.ds(h * 64, 64)], gsem.at[b]).wait()
        for sub in range(NSC):
            pltpu.make_async_copy(er_hbm.at[idx_d.at[0, 0, 0]],
                                  errs.at[b, pl.ds(sub * SCH, SCH)],
                                  gsem.at[b]).wait()

    def start_scatter(slot, row, sub, p):
        pltpu.async_copy(sbuf.at[p], acc.at[idx_d.at[slot, row, sub]],
                         ssem.at[p], add=True)

    def wait_scatter(p):
        pltpu.make_async_copy(sbuf.at[p], acc.at[idx_d.at[0, 0, 0]],
                              ssem.at[p]).wait()

    def unpack2(vi):
        va = lax.bitcast_convert_type(lax.shift_left(vi, 16), jnp.float32)
        vb = lax.bitcast_convert_type(
            jnp.bitwise_and(vi, jnp.int32(-65536)), jnp.float32)
        return va, vb

    def compute(b, p, sub):
        @plsc.parallel_loop(sub * SCH, (sub + 1) * SCH, unroll=4)
        def edge_body(e):
            es = e - sub * SCH
            elr, _ = unpack2(rows[b, 4, e, :])
            erd, _ = unpack2(errs[b, e, :])
            ev = elr + erd
            ev = jnp.maximum(ev, 0.2 * ev)
            wv = jnp.exp(ev)
            sbuf[p, es, pl.ds(D, 16)] = wv
            for k in range(4):
                ha, hb = unpack2(rows[b, k, e, :])
                ba = lax.broadcast(wv[2 * k], (16,))
                bb = lax.broadcast(wv[2 * k + 1], (16,))
                sbuf[p, es, pl.ds(F * 2 * k, F)] = ha * ba
                sbuf[p, es, pl.ds(F * (2 * k + 1), F)] = hb * bb

    pltpu.sync_copy(src_hbm.at[pl.ds(w0, ISUP)], idx_s.at[0])
    pltpu.sync_copy(dst_hbm.at[pl.ds(w0, ISUP)], idx_d.at[0])
    start_gather(0, 0, 0)
    start_gather(0, 1, 1)

    def chunk_body(i, carry):
        j = i // ISUP
        t = i - j * ISUP
        cs = j % 2
        ns = 1 - cs
        b = i % NBUF
        b2 = (i + 2) % NBUF
        more_sups = j < NSUP - 1

        @pl.when(jnp.logical_and(t == 1, more_sups))
        def _():
            load_idx(j + 1, ns)

        wait_gather(b)

        for sub in range(NSC):
            ps = sub % SBUF
            if sub < SBUF:
                @pl.when(i > 0)
                def _(ps=ps):
                    wait_scatter(ps)
            else:
                wait_scatter(ps)
            compute(b, ps, sub)
            start_scatter(cs, t, sub, ps)

        @pl.when(jnp.logical_and(t == ISUP - 2, more_sups))
        def _():
            wait_idx(ns)

        tp = t + 2
        wrap = tp >= ISUP
        g_slot = jnp.where(wrap, ns, cs)
        g_row = jnp.where(wrap, tp - ISUP, tp)

        @pl.when(jnp.logical_or(jnp.logical_not(wrap), more_sups))
        def _():
            start_gather(g_slot, g_row, b2)

        return carry

    lax.fori_loop(0, CHUNKS, chunk_body, 0)
    wait_scatter(0)
    wait_scatter(1)
    plsc.subcore_barrier()
    pltpu.sync_copy(acc.at[pl.ds(s * ROWS_PER_TILE, ROWS_PER_TILE)],
                    out_hbm.at[c, pl.ds(s * ROWS_PER_TILE, ROWS_PER_TILE)])


def _block_diag(a):
    eye = jnp.eye(H, dtype=jnp.float32)
    return (a[:, :, None] * eye[:, None, :]).reshape(D, H)


def _tables_weights(al, ar):
    alr = jnp.concatenate([_block_diag(al), _block_diag(ar)], axis=1)
    i_alr = jnp.concatenate([jnp.eye(D, dtype=jnp.float32), alr], axis=1)
    g = i_alr @ jnp.asarray(_P)
    arr = jnp.concatenate([_block_diag(ar), _block_diag(ar)], axis=1)
    ger = arr @ jnp.asarray(_Q)
    return g, ger


def kernel(n_feat, edge_index, e_feat, W1, al1, ar1, b1, W2, al2, ar2, b2):
    del e_feat
    ei = edge_index.astype(jnp.int32)
    pad_e = E_PAD - E_RAW
    src = jnp.concatenate([ei[0], jnp.zeros((pad_e,), jnp.int32)])
    src = src.reshape(E_PAD // CH, CH)
    dst = jnp.concatenate([ei[1], jnp.full((pad_e,), N, jnp.int32)])
    dst = dst.reshape(E_PAD // CH, NSC, SCH)
    zero_blk = jnp.zeros((ROWS_PER_TILE, TW), jnp.float32)
    rep = jnp.repeat(jnp.eye(H, dtype=jnp.float32), F, axis=1)

    g1, ger1 = _tables_weights(al1, ar1)
    g2, ger2 = _tables_weights(al2, ar2)

    def as_i32(t):
        return lax.bitcast_convert_type(
            t.reshape(t.shape[0], t.shape[1] // 2, 2), jnp.int32)

    x = jnp.pad(n_feat, ((0, N_ACC - N), (0, 0)))
    t1, er1 = _embed(x, W1, g1, ger1, rows_blk=2528)
    sc_edge = _make_sc_edge()
    t1i = as_i32(t1)
    acc1 = sc_edge(*[t1i[:, 16 * g:16 * g + 16] for g in range(5)],
                   as_i32(er1), src, dst, zero_blk)
    t2, er2 = _norm_embed(acc1, rep, b1.reshape(1, D), W2, g2, ger2,
                          rows_blk=2528)
    t2i = as_i32(t2)
    acc2 = sc_edge(*[t2i[:, 16 * g:16 * g + 16] for g in range(5)],
                   as_i32(er2), src, dst, zero_blk)
    return _final(acc2, rep, b2.reshape(1, D), rows_blk=2000)


# speedup vs baseline: 3.0475x; 1.3085x over previous
import functools

import jax
import jax.numpy as jnp
import numpy as np
from jax import lax
from jax.experimental import pallas as pl
from jax.experimental.pallas import tpu as pltpu
from jax.experimental.pallas import tpu_sc as plsc

N = 10000
D = 128
H = 8
F = 16
TW = D + 2 * H
TM = 160
N_CORES = 2
N_SUB = 16
N_WORK = N_CORES * N_SUB
CH = 128
SCH = 32
NSC = CH // SCH
E_RAW = 320000
NBUF = 2
SBUF = 2
ISUP = 9
NSUP = 9
CHUNKS = ISUP * NSUP
E_PAD = N_WORK * CH * CHUNKS
EPW = CH * CHUNKS
N_ACC = 10112
ROWS_PER_TILE = N_ACC // N_SUB


def _bits(x):
    return jax.lax.bitcast_convert_type(
        x.astype(jnp.bfloat16).astype(jnp.float32), jnp.int32)


def _pack_tables(h, ga, gb, ger, t_refs, er_ref):
    pa = _bits(jnp.dot(h, ga, preferred_element_type=jnp.float32))
    pb = _bits(jnp.dot(h, gb, preferred_element_type=jnp.float32))
    packed = jax.lax.shift_right_logical(pa, 16) | pb
    for g in range(5):
        t_refs[g][...] = packed[:, 16 * g:16 * (g + 1)]
    pe = _bits(jnp.dot(h, ger, preferred_element_type=jnp.float32))
    er_ref[...] = jax.lax.shift_right_logical(pe, 16) | pe


def _embed_body(x_ref, w_ref, ga_ref, gb_ref, ger_ref, t0, t1, t2, t3, t4,
                er_ref):
    h = jnp.dot(x_ref[...], w_ref[...], preferred_element_type=jnp.float32)
    _pack_tables(h, ga_ref[...], gb_ref[...], ger_ref[...],
                 (t0, t1, t2, t3, t4), er_ref)


_TAB_OUT_SPECS = [pl.BlockSpec((2528, 16), lambda i: (i, 0))
                  for _ in range(6)]
_TAB_OUT_SHAPES = [jax.ShapeDtypeStruct((N_ACC, 16), jnp.int32)
                   for _ in range(6)]


def _embed(x, w, ga, gb, ger, rows_blk):
    n = x.shape[0]
    return pl.pallas_call(
        _embed_body,
        grid=(n // rows_blk,),
        in_specs=[
            pl.BlockSpec((rows_blk, D), lambda i: (i, 0)),
            pl.BlockSpec((D, D), lambda i: (0, 0)),
            pl.BlockSpec((D, 80), lambda i: (0, 0)),
            pl.BlockSpec((D, 80), lambda i: (0, 0)),
            pl.BlockSpec((D, 2 * H), lambda i: (0, 0)),
        ],
        out_specs=_TAB_OUT_SPECS,
        out_shape=_TAB_OUT_SHAPES,
    )(x, w, ga, gb, ger)


def _norm_embed_body(acc_ref, rep_ref, b_ref, w_ref, ga_ref, gb_ref, ger_ref,
                     t0, t1, t2, t3, t4, er_ref):
    s = acc_ref[0] + acc_ref[1]
    den = jnp.dot(s[:, D:D + H], rep_ref[...], preferred_element_type=jnp.float32)
    x1 = s[:, :D] / (den + 1e-9) + b_ref[...]
    x1 = jnp.maximum(x1, 0.01 * x1)
    h = jnp.dot(x1, w_ref[...], preferred_element_type=jnp.float32)
    _pack_tables(h, ga_ref[...], gb_ref[...], ger_ref[...],
                 (t0, t1, t2, t3, t4), er_ref)


def _norm_embed(acc, rep, b, w, ga, gb, ger, rows_blk):
    n = acc.shape[1]
    return pl.pallas_call(
        _norm_embed_body,
        grid=(n // rows_blk,),
        in_specs=[
            pl.BlockSpec((2, rows_blk, TW), lambda i: (0, i, 0)),
            pl.BlockSpec((H, D), lambda i: (0, 0)),
            pl.BlockSpec((1, D), lambda i: (0, 0)),
            pl.BlockSpec((D, D), lambda i: (0, 0)),
            pl.BlockSpec((D, 80), lambda i: (0, 0)),
            pl.BlockSpec((D, 80), lambda i: (0, 0)),
            pl.BlockSpec((D, 2 * H), lambda i: (0, 0)),
        ],
        out_specs=_TAB_OUT_SPECS,
        out_shape=_TAB_OUT_SHAPES,
    )(acc, rep, b, w, ga, gb, ger)


def _final_body(acc_ref, rep_ref, b_ref, o_ref):
    s = acc_ref[0] + acc_ref[1]
    den = jnp.dot(s[:, D:D + H], rep_ref[...], preferred_element_type=jnp.float32)
    o_ref[...] = s[:, :D] / (den + 1e-9) + b_ref[...]


def _final(acc, rep, b, rows_blk):
    return pl.pallas_call(
        _final_body,
        grid=(N // rows_blk,),
        in_specs=[
            pl.BlockSpec((2, rows_blk, TW), lambda i: (0, i, 0)),
            pl.BlockSpec((H, D), lambda i: (0, 0)),
            pl.BlockSpec((1, D), lambda i: (0, 0)),
        ],
        out_specs=pl.BlockSpec((rows_blk, D), lambda i: (i, 0)),
        out_shape=jax.ShapeDtypeStruct((N, D), jnp.float32),
    )(acc, rep, b)


@functools.cache
def _make_sc_edge():
    mesh = plsc.VectorSubcoreMesh(core_axis_name="c", subcore_axis_name="s")
    return functools.partial(
        pl.kernel,
        mesh=mesh,
        compiler_params=pltpu.CompilerParams(use_tc_tiling_on_sc=False),
        out_type=jax.ShapeDtypeStruct((N_CORES, N_ACC, TW), jnp.float32),
        scratch_types=[
            pltpu.VMEM((2, ISUP, CH), jnp.int32),
            pltpu.VMEM((2, ISUP, NSC, SCH), jnp.int32),
            pltpu.VMEM((NBUF, 5, CH, 2 * H), jnp.int32),
            pltpu.VMEM((NBUF, CH, 2 * H), jnp.int32),
            pltpu.VMEM((SBUF, SCH, TW), jnp.float32),
            pltpu.VMEM_SHARED((N_ACC, TW), jnp.float32),
            pltpu.SemaphoreType.DMA((NBUF,)),
            pltpu.SemaphoreType.DMA((SBUF,)),
            pltpu.SemaphoreType.DMA,
        ],
    )(_sc_edge_body)


def _sc_edge_body(t0_hbm, t1_hbm, t2_hbm, t3_hbm, t4_hbm, er_hbm, src_hbm,
                  dst_hbm, zero_hbm, out_hbm, idx_s, idx_d, rows, errs, sbuf,
                  acc, gsem, ssem, isem):
    c = lax.axis_index("c")
    s = lax.axis_index("s")
    pltpu.sync_copy(zero_hbm, acc.at[pl.ds(s * ROWS_PER_TILE, ROWS_PER_TILE)])
    plsc.subcore_barrier()

    w0 = (c * N_SUB + s) * CHUNKS

    def load_idx(sup, slot):
        pltpu.async_copy(src_hbm.at[pl.ds(w0 + sup * ISUP, ISUP)],
                         idx_s.at[slot], isem)
        pltpu.async_copy(dst_hbm.at[pl.ds(w0 + sup * ISUP, ISUP)],
                         idx_d.at[slot], isem)

    def wait_idx(slot):
        pltpu.make_async_copy(src_hbm.at[pl.ds(w0, ISUP)], idx_s.at[slot],
                              isem).wait()
        pltpu.make_async_copy(dst_hbm.at[pl.ds(w0, ISUP)], idx_d.at[slot],
                              isem).wait()

    t_tabs = (t0_hbm, t1_hbm, t2_hbm, t3_hbm, t4_hbm)

    def start_gather(slot, row, b):
        for g in range(5):
            for h in range(2):
                pltpu.async_copy(
                    t_tabs[g].at[idx_s.at[slot, row, pl.ds(h * 64, 64)]],
                    rows.at[b, g, pl.ds(h * 64, 64)], gsem.at[b])
        for sub in range(NSC):
            pltpu.async_copy(er_hbm.at[idx_d.at[slot, row, sub]],
                             errs.at[b, pl.ds(sub * SCH, SCH)], gsem.at[b])

    def wait_gather(b):
        for g in range(5):
            for h in range(2):
                pltpu.make_async_copy(
                    t_tabs[g].at[idx_s.at[0, 0, pl.ds(h * 64, 64)]],
                    rows.at[b, g, pl.ds(h * 64, 64)], gsem.at[b]).wait()
        for sub in range(NSC):
            pltpu.make_async_copy(er_hbm.at[idx_d.at[0, 0, 0]],
                                  errs.at[b, pl.ds(sub * SCH, SCH)],
                                  gsem.at[b]).wait()

    def start_scatter(slot, row, sub, p):
        pltpu.async_copy(sbuf.at[p], acc.at[idx_d.at[slot, row, sub]],
                         ssem.at[p], add=True)

    def wait_scatter(p):
        pltpu.make_async_copy(sbuf.at[p], acc.at[idx_d.at[0, 0, 0]],
                              ssem.at[p]).wait()

    def unpack2(vi):
        va = lax.bitcast_convert_type(lax.shift_left(vi, 16), jnp.float32)
        vb = lax.bitcast_convert_type(
            jnp.bitwise_and(vi, jnp.int32(-65536)), jnp.float32)
        return va, vb

    def compute(b, p, sub):
        @plsc.parallel_loop(sub * SCH, (sub + 1) * SCH, unroll=4)
        def edge_body(e):
            es = e - sub * SCH
            elr, _ = unpack2(rows[b, 4, e, :])
            erd, _ = unpack2(errs[b, e, :])
            ev = elr + erd
            ev = jnp.maximum(ev, 0.2 * ev)
            wv = jnp.exp(ev)
            sbuf[p, es, pl.ds(D, 16)] = wv
            for k in range(4):
                ha, hb = unpack2(rows[b, k, e, :])
                ba = lax.broadcast(wv[2 * k], (16,))
                bb = lax.broadcast(wv[2 * k + 1], (16,))
                sbuf[p, es, pl.ds(F * 2 * k, F)] = ha * ba
                sbuf[p, es, pl.ds(F * (2 * k + 1), F)] = hb * bb

    pltpu.sync_copy(src_hbm.at[pl.ds(w0, ISUP)], idx_s.at[0])
    pltpu.sync_copy(dst_hbm.at[pl.ds(w0, ISUP)], idx_d.at[0])
    start_gather(0, 0, 0)
    start_gather(0, 1, 1)

    def chunk_body(i, carry):
        j = i // ISUP
        t = i - j * ISUP
        cs = j % 2
        ns = 1 - cs
        b = i % NBUF
        b2 = (i + 2) % NBUF
        more_sups = j < NSUP - 1

        @pl.when(jnp.logical_and(t == 1, more_sups))
        def _():
            load_idx(j + 1, ns)

        wait_gather(b)

        for sub in range(NSC):
            ps = sub % SBUF
            if sub < SBUF:
                @pl.when(i > 0)
                def _(ps=ps):
                    wait_scatter(ps)
            else:
                wait_scatter(ps)
            compute(b, ps, sub)
            start_scatter(cs, t, sub, ps)

        @pl.when(jnp.logical_and(t == ISUP - 2, more_sups))
        def _():
            wait_idx(ns)

        tp = t + 2
        wrap = tp >= ISUP
        g_slot = jnp.where(wrap, ns, cs)
        g_row = jnp.where(wrap, tp - ISUP, tp)

        @pl.when(jnp.logical_or(jnp.logical_not(wrap), more_sups))
        def _():
            start_gather(g_slot, g_row, b2)

        return carry

    lax.fori_loop(0, CHUNKS, chunk_body, 0)
    wait_scatter(0)
    wait_scatter(1)
    plsc.subcore_barrier()
    pltpu.sync_copy(acc.at[pl.ds(s * ROWS_PER_TILE, ROWS_PER_TILE)],
                    out_hbm.at[c, pl.ds(s * ROWS_PER_TILE, ROWS_PER_TILE)])


def _block_diag(a):
    eye = jnp.eye(H, dtype=jnp.float32)
    return (a[:, :, None] * eye[:, None, :]).reshape(D, H)


_COLS_A = np.array([32 * k + j for k in range(4) for j in range(16)]
                   + [D + j for j in range(16)])
_COLS_B = np.array([32 * k + 16 + j for k in range(4) for j in range(16)]
                   + [D + j for j in range(16)])


def _tables_weights(al, ar):
    alr = jnp.concatenate([_block_diag(al), _block_diag(ar)], axis=1)
    i_alr = jnp.concatenate([jnp.eye(D, dtype=jnp.float32), alr], axis=1)
    ger = jnp.concatenate([_block_diag(ar), _block_diag(ar)], axis=1)
    return i_alr[:, _COLS_A], i_alr[:, _COLS_B], ger


def kernel(n_feat, edge_index, e_feat, W1, al1, ar1, b1, W2, al2, ar2, b2):
    del e_feat
    ei = edge_index.astype(jnp.int32)
    pad_e = E_PAD - E_RAW
    src = jnp.concatenate([ei[0], jnp.zeros((pad_e,), jnp.int32)])
    src = src.reshape(E_PAD // CH, CH)
    dst = jnp.concatenate([ei[1], jnp.full((pad_e,), N, jnp.int32)])
    dst = dst.reshape(E_PAD // CH, NSC, SCH)
    zero_blk = jnp.zeros((ROWS_PER_TILE, TW), jnp.float32)
    rep = jnp.repeat(jnp.eye(H, dtype=jnp.float32), F, axis=1)

    ga1, gb1, ger1 = _tables_weights(al1, ar1)
    ga2, gb2, ger2 = _tables_weights(al2, ar2)

    x = jnp.pad(n_feat, ((0, N_ACC - N), (0, 0)))
    tabs1 = _embed(x, W1, ga1, gb1, ger1, rows_blk=2528)
    sc_edge = _make_sc_edge()
    acc1 = sc_edge(*tabs1, src, dst, zero_blk)
    tabs2 = _norm_embed(acc1, rep, b1.reshape(1, D), W2, ga2, gb2, ger2,
                        rows_blk=2528)
    acc2 = sc_edge(*tabs2, src, dst, zero_blk)
    return _final(acc2, rep, b2.reshape(1, D), rows_blk=2000)
